# baseline (speedup 1.0000x reference)
import functools

import jax
import jax.numpy as jnp
from jax.experimental import pallas as pl
from jax.experimental.pallas import tpu as pltpu


_TB = 2048


def _shrink_kernel(x_ref, slot_ref, a_ref, h_ref, *, rank_shift):
    xb = x_ref[...].astype(jnp.bfloat16)
    h = jnp.dot(xb, a_ref[...], preferred_element_type=jnp.float32)
    slot = slot_ref[0, 0, :]
    eidx = jax.lax.broadcasted_iota(jnp.int32, h.shape, 1) >> rank_shift
    h_ref[...] = jnp.where(eidx == slot[:, None], h, 0.0).astype(jnp.bfloat16)


def _expand_kernel(h_ref, b_ref, o_ref, b_scr):
    @pl.when(pl.program_id(0) == 0)
    def _prep():
        b_scr[...] = b_ref[...].astype(jnp.bfloat16)

    o_ref[...] = jnp.dot(h_ref[...], b_scr[...], preferred_element_type=jnp.float32)


def kernel(x, token_to_slot, lora_a, lora_b, lora_scaling):
    T, D = x.shape
    E, _, R = lora_a.shape
    D_OUT = lora_b.shape[-1]
    assert R & (R - 1) == 0
    rank_shift = R.bit_length() - 1

    a_all = (
        (lora_a * lora_scaling[:, None, None])
        .transpose(1, 0, 2)
        .reshape(D, E * R)
        .astype(jnp.bfloat16)
    )
    b2 = lora_b.reshape(E * R, D_OUT)
    n_t = T // _TB
    slot3 = token_to_slot.reshape(n_t, 1, _TB)

    hm = pl.pallas_call(
        functools.partial(_shrink_kernel, rank_shift=rank_shift),
        grid=(n_t,),
        in_specs=[
            pl.BlockSpec((_TB, D), lambda i: (i, 0)),
            pl.BlockSpec((1, 1, _TB), lambda i: (i, 0, 0)),
            pl.BlockSpec((D, E * R), lambda i: (0, 0)),
        ],
        out_specs=pl.BlockSpec((_TB, E * R), lambda i: (i, 0)),
        out_shape=jax.ShapeDtypeStruct((T, E * R), jnp.bfloat16),
    )(x, slot3, a_all)

    return pl.pallas_call(
        _expand_kernel,
        grid=(n_t,),
        in_specs=[
            pl.BlockSpec((_TB, E * R), lambda i: (i, 0)),
            pl.BlockSpec((E * R, D_OUT), lambda i: (0, 0)),
        ],
        out_specs=pl.BlockSpec((_TB, D_OUT), lambda i: (i, 0)),
        out_shape=jax.ShapeDtypeStruct((T, D_OUT), x.dtype),
        scratch_shapes=[pltpu.VMEM((E * R, D_OUT), jnp.bfloat16)],
    )(hm, b2)


# speedup vs baseline: 1.0207x; 1.0207x over previous
import functools

import jax
import jax.numpy as jnp
from jax.experimental import pallas as pl
from jax.experimental.pallas import tpu as pltpu


_TB = 1024


def _shrink_kernel(x_ref, slot_ref, a_ref, h_ref, *, rank_shift):
    xb = x_ref[...].astype(jnp.bfloat16)
    h = jnp.dot(xb, a_ref[...], preferred_element_type=jnp.float32)
    slot = slot_ref[0, 0, :]
    eidx = jax.lax.broadcasted_iota(jnp.int32, h.shape, 1) >> rank_shift
    h_ref[...] = jnp.where(eidx == slot[:, None], h, 0.0).astype(jnp.bfloat16)


def _expand_kernel(h_ref, b_ref, o_ref, b_scr):
    @pl.when(pl.program_id(0) == 0)
    def _prep():
        b_scr[...] = b_ref[...].astype(jnp.bfloat16)

    o_ref[...] = jnp.dot(h_ref[...], b_scr[...], preferred_element_type=jnp.float32)


def kernel(x, token_to_slot, lora_a, lora_b, lora_scaling):
    T, D = x.shape
    E, _, R = lora_a.shape
    D_OUT = lora_b.shape[-1]
    assert R & (R - 1) == 0
    rank_shift = R.bit_length() - 1

    a_all = (
        (lora_a * lora_scaling[:, None, None])
        .transpose(1, 0, 2)
        .reshape(D, E * R)
        .astype(jnp.bfloat16)
    )
    b2 = lora_b.reshape(E * R, D_OUT)
    n_t = T // _TB
    slot3 = token_to_slot.reshape(n_t, 1, _TB)

    hm = pl.pallas_call(
        functools.partial(_shrink_kernel, rank_shift=rank_shift),
        grid=(n_t,),
        in_specs=[
            pl.BlockSpec((_TB, D), lambda i: (i, 0)),
            pl.BlockSpec((1, 1, _TB), lambda i: (i, 0, 0)),
            pl.BlockSpec((D, E * R), lambda i: (0, 0)),
        ],
        out_specs=pl.BlockSpec((_TB, E * R), lambda i: (i, 0)),
        out_shape=jax.ShapeDtypeStruct((T, E * R), jnp.bfloat16),
    )(x, slot3, a_all)

    return pl.pallas_call(
        _expand_kernel,
        grid=(n_t,),
        in_specs=[
            pl.BlockSpec((_TB, E * R), lambda i: (i, 0)),
            pl.BlockSpec((E * R, D_OUT), lambda i: (0, 0)),
        ],
        out_specs=pl.BlockSpec((_TB, D_OUT), lambda i: (i, 0)),
        out_shape=jax.ShapeDtypeStruct((T, D_OUT), x.dtype),
        scratch_shapes=[pltpu.VMEM((E * R, D_OUT), jnp.bfloat16)],
    )(hm, b2)
